# Initial kernel scaffold; baseline (speedup 1.0000x reference)
import jax
import jax.numpy as jnp
from jax.experimental import pallas as pl


def kernel(point_clouds, params):
    raise NotImplementedError("write your pallas kernel here")


# speedup vs baseline: 32.3487x; 32.3487x over previous
import functools

import numpy as np

import jax
import jax.numpy as jnp
from jax.experimental import pallas as pl
from jax.experimental.pallas import tpu as pltpu
from jax.experimental.pallas import tpu_sc as plsc

BN_EPS = 1e-5


_SQC = float(np.sqrt(np.float32(1.0 + BN_EPS)))


def _rb16(v):
    return v.astype(jnp.bfloat16).astype(jnp.float32)


def _fps_call(xp, yp, zp, npoint):
    B, Npad = xp.shape

    def body(x_ref, y_ref, z_ref, cx_ref, cy_ref, cz_ref):
        x = x_ref[...]
        y = y_ref[...]
        z = z_ref[...]
        iota = jax.lax.broadcasted_iota(jnp.int32, (B, Npad), 1)
        iota_np = jax.lax.broadcasted_iota(jnp.int32, (B, npoint), 1)

        def step(i, carry):
            dist, far, cxs, cys, czs = carry
            oh = (iota == far).astype(jnp.float32)
            cx = jnp.sum(x * oh, axis=1, keepdims=True)
            cy = jnp.sum(y * oh, axis=1, keepdims=True)
            cz = jnp.sum(z * oh, axis=1, keepdims=True)
            sel = iota_np == i
            cxs = jnp.where(sel, cx, cxs)
            cys = jnp.where(sel, cy, cys)
            czs = jnp.where(sel, cz, czs)
            dx = x - cx
            dy = y - cy
            dz = z - cz
            d = dx * dx + dy * dy + dz * dz
            dist = jnp.minimum(dist, d)
            mx = jnp.max(dist, axis=1, keepdims=True)
            far = jnp.min(jnp.where(dist == mx, iota, Npad),
                          axis=1, keepdims=True).astype(jnp.int32)
            return dist, far, cxs, cys, czs

        zc = jnp.zeros((B, npoint), jnp.float32)
        init = (jnp.full((B, Npad), 1e10, jnp.float32),
                jnp.zeros((B, 1), jnp.int32), zc, zc, zc)
        _, _, cxs, cys, czs = jax.lax.fori_loop(0, npoint, step, init)
        cx_ref[...] = cxs
        cy_ref[...] = cys
        cz_ref[...] = czs

    out = [jax.ShapeDtypeStruct((B, npoint), jnp.float32)] * 3
    return pl.pallas_call(body, out_shape=out)(xp, yp, zp)


def _bq_call(xp, yp, zp, cx, cy, cz, radius, nsample, n_real):
    B, Npad = xp.shape
    Spad = cx.shape[1]
    SB = 128
    r2 = radius ** 2

    def body(x_ref, y_ref, z_ref, cx_ref, cy_ref, cz_ref, o_ref):
        x = x_ref[0]
        y = y_ref[0]
        z = z_ref[0]
        a = cx_ref[0]
        b = cy_ref[0]
        c = cz_ref[0]
        xr = _rb16(x)
        yr = _rb16(y)
        zr = _rb16(z)
        ar = _rb16(a)
        br = _rb16(b)
        cr = _rb16(c)
        m = ar * xr + br * yr
        m = m + cr * zr
        d = -2.0 * m
        d = d + ((a * a + b * b) + c * c)
        d = d + ((x * x + y * y) + z * z)
        iota = jax.lax.broadcasted_iota(jnp.int32, (SB, Npad), 1)
        cand = jnp.where(d <= r2, iota, n_real)
        cols = []
        for _ in range(nsample):
            mn = jnp.min(cand, axis=1, keepdims=True)
            cols.append(mn)
            cand = jnp.where(cand == mn, n_real, cand)
        idx = jnp.concatenate(cols, axis=1)
        first = idx[:, 0:1]
        idx = jnp.where(idx == n_real, first, idx)
        o_ref[0] = idx

    return pl.pallas_call(
        body,
        grid=(B, Spad // SB),
        in_specs=[pl.BlockSpec((1, 1, Npad), lambda bb, s: (bb, 0, 0))] * 3
        + [pl.BlockSpec((1, SB, 1), lambda bb, s: (bb, s, 0))] * 3,
        out_specs=pl.BlockSpec((1, SB, nsample), lambda bb, s: (bb, s, 0)),
        out_shape=jax.ShapeDtypeStruct((B, Spad, nsample), jnp.int32),
    )(xp.reshape(B, 1, Npad), yp.reshape(B, 1, Npad), zp.reshape(B, 1, Npad),
      cx, cy, cz)


def _sc_gather(table, idx_flat):
    num = idx_flat.shape[0]
    vdim = table.shape[1]
    nchunks = num // 128
    nw = 32
    mesh = plsc.VectorSubcoreMesh(core_axis_name="c", subcore_axis_name="s")

    @functools.partial(
        pl.kernel, mesh=mesh,
        out_type=jax.ShapeDtypeStruct((num, vdim), table.dtype),
        scratch_types=[pltpu.VMEM((128,), jnp.int32),
                       pltpu.VMEM((128, vdim), table.dtype)],
    )
    def k(table_hbm, idx_hbm, out_hbm, idx_v, rows_v):
        wid = jax.lax.axis_index("s") * 2 + jax.lax.axis_index("c")

        @pl.loop(wid, nchunks, step=nw)
        def _(c):
            pltpu.sync_copy(idx_hbm.at[pl.ds(c * 128, 128)], idx_v)
            pltpu.sync_copy(table_hbm.at[idx_v], rows_v)
            pltpu.sync_copy(rows_v, out_hbm.at[pl.ds(c * 128, 128)])

    return k(table, idx_flat)


def _sa_mlp_call(g, cen, layers, s_count, ns):
    B, _, cpad = g.shape
    cout = layers[-1][0].shape[1]
    nlay = len(layers)

    def body(*refs):
        g_ref, c_ref = refs[0], refs[1]
        wrefs = refs[2:2 + 4 * nlay]
        o_ref = refs[2 + 4 * nlay]
        x = g_ref[0]
        c = c_ref[0]
        x = x.reshape(s_count, ns, cpad) - c[:, None, :]
        x = x.reshape(s_count * ns, cpad)
        for i in range(nlay):
            w = wrefs[4 * i][...]
            bb = wrefs[4 * i + 1][...]
            ga = wrefs[4 * i + 2][...]
            be = wrefs[4 * i + 3][...]
            y = jnp.dot(x.astype(jnp.bfloat16), w.astype(jnp.bfloat16),
                        preferred_element_type=jnp.float32) + bb
            x = jnp.maximum(ga * y / _SQC + be, 0.0)
        o_ref[0] = jnp.max(x.reshape(s_count, ns, cout), axis=1)

    w_specs = []
    w_args = []
    for lay in layers:
        for arr in lay:
            w_specs.append(pl.BlockSpec(arr.shape, lambda b_: (0, 0)))
            w_args.append(arr)

    return pl.pallas_call(
        body,
        grid=(B,),
        in_specs=[pl.BlockSpec((1, s_count * ns, cpad), lambda b_: (b_, 0, 0)),
                  pl.BlockSpec((1, s_count, cpad), lambda b_: (b_, 0, 0))]
        + w_specs,
        out_specs=pl.BlockSpec((1, s_count, cout), lambda b_: (b_, 0, 0)),
        out_shape=jax.ShapeDtypeStruct((B, s_count, cout), jnp.float32),
    )(g, cen, *w_args)


def _fp_call(x1, x2, p2, p1, layers, relu_flags, bn_flags, nblk):
    B, n1pad, _ = x1[0].shape
    spad = x2[0].shape[2]
    nlay = len(layers)
    cout = layers[-1][0].shape[1]
    has_p1 = p1 is not None

    def body(*refs):
        i = 0
        ax_ref, ay_ref, az_ref = refs[0:3]
        bx_ref, by_ref, bz_ref = refs[3:6]
        p2_ref = refs[6]
        i = 7
        p1_ref = None
        if has_p1:
            p1_ref = refs[7]
            i = 8
        wrefs = refs[i:i + 4 * nlay]
        o_ref = refs[i + 4 * nlay]

        ax = ax_ref[0]
        ay = ay_ref[0]
        az = az_ref[0]
        bx = bx_ref[0]
        by = by_ref[0]
        bz = bz_ref[0]
        axr = _rb16(ax)
        ayr = _rb16(ay)
        azr = _rb16(az)
        bxr = _rb16(bx)
        byr = _rb16(by)
        bzr = _rb16(bz)
        m = axr * bxr + ayr * byr
        m = m + azr * bzr
        d = -2.0 * m
        d = d + ((ax * ax + ay * ay) + az * az)
        d = d + ((bx * bx + by * by) + bz * bz)

        iota = jax.lax.broadcasted_iota(jnp.int32, (nblk, spad), 1)
        cand = d
        norm = jnp.zeros((nblk, 1), jnp.float32)
        wsum = jnp.zeros((nblk, spad), jnp.float32)
        for _ in range(3):
            dk = jnp.min(cand, axis=1, keepdims=True)
            ik = jnp.min(jnp.where(cand == dk, iota, spad),
                         axis=1, keepdims=True).astype(jnp.int32)
            oh = (iota == ik).astype(jnp.float32)
            recip = 1.0 / (dk + 1e-8)
            norm = norm + recip
            wsum = wsum + oh * recip
            cand = jnp.where(oh > 0.0, jnp.inf, cand)
        wmat = wsum / norm
        interp = jnp.dot(wmat, p2_ref[0], preferred_element_type=jnp.float32,
                         precision=jax.lax.Precision.HIGHEST)
        if has_p1:
            x = jnp.concatenate([p1_ref[0], interp], axis=1)
        else:
            x = interp
        for li in range(nlay):
            w = wrefs[4 * li][...]
            bb = wrefs[4 * li + 1][...]
            ga = wrefs[4 * li + 2][...]
            be = wrefs[4 * li + 3][...]
            x = jnp.dot(x.astype(jnp.bfloat16), w.astype(jnp.bfloat16),
                        preferred_element_type=jnp.float32) + bb
            if bn_flags[li]:
                x = ga * x / _SQC + be
            if relu_flags[li]:
                x = jnp.maximum(x, 0.0)
        o_ref[0] = x

    in_specs = [pl.BlockSpec((1, nblk, 1), lambda b_, nb: (b_, nb, 0))] * 3
    in_specs += [pl.BlockSpec((1, 1, spad), lambda b_, nb: (b_, 0, 0))] * 3
    in_specs += [pl.BlockSpec((1, spad, p2.shape[2]),
                              lambda b_, nb: (b_, 0, 0))]
    args = list(x1) + list(x2) + [p2]
    if has_p1:
        in_specs += [pl.BlockSpec((1, nblk, p1.shape[2]),
                                  lambda b_, nb: (b_, nb, 0))]
        args.append(p1)
    w_args = []
    for lay in layers:
        for arr in lay:
            in_specs.append(pl.BlockSpec(arr.shape, lambda b_, nb: (0, 0)))
            w_args.append(arr)

    return pl.pallas_call(
        body,
        grid=(B, n1pad // nblk),
        in_specs=in_specs,
        out_specs=pl.BlockSpec((1, nblk, cout), lambda b_, nb: (b_, nb, 0)),
        out_shape=jax.ShapeDtypeStruct((B, n1pad, cout), jnp.float32),
    )(*args, *w_args)


def _prep_layer(layer, cpad=None):
    w, b, g, be = layer
    wt = w.T
    if cpad is not None and wt.shape[0] < cpad:
        wt = jnp.pad(wt, ((0, cpad - wt.shape[0]), (0, 0)))
    return wt, b[None, :], g[None, :], be[None, :]


def _pad_lanes(a, npad, mode):
    b, n = a.shape
    if npad == n:
        return a
    if mode == "dup0":
        fill = jnp.broadcast_to(a[:, 0:1], (b, npad - n))
    else:
        fill = jnp.full((b, npad - n), 1e9, a.dtype)
    return jnp.concatenate([a, fill], axis=1)


def _pad_rows(a, npad, value=0.0):
    if a.shape[1] == npad:
        return a
    return jnp.pad(a, ((0, 0), (0, npad - a.shape[1]), (0, 0)),
                   constant_values=value)


def _set_abstraction(cx, cy, cz, feats, xyz_cols, npoint, radius, ns,
                     layers_raw, n_real, npad, spad, cpad):
    b = cx.shape[0]
    fx = _pad_lanes(cx, npad, "dup0")
    fy = _pad_lanes(cy, npad, "dup0")
    fz = _pad_lanes(cz, npad, "dup0")
    ncx, ncy, ncz = _fps_call(fx, fy, fz, npoint)
    qx = _pad_lanes(cx, npad, "far")
    qy = _pad_lanes(cy, npad, "far")
    qz = _pad_lanes(cz, npad, "far")
    ccx = _pad_lanes(ncx, spad, "far")[:, :, None]
    ccy = _pad_lanes(ncy, spad, "far")[:, :, None]
    ccz = _pad_lanes(ncz, spad, "far")[:, :, None]
    idx = _bq_call(qx, qy, qz, ccx, ccy, ccz, radius, ns, n_real)
    idx = idx[:, :npoint, :]
    ctab = xyz_cols.shape[2]
    table = xyz_cols
    if ctab < cpad:
        table = jnp.pad(table, ((0, 0), (0, 0), (0, cpad - ctab)))
    table = table.reshape(b * n_real, cpad)
    idx_flat = (idx + (jnp.arange(b, dtype=jnp.int32) * n_real)[:, None, None])
    idx_flat = idx_flat.reshape(-1)
    gathered = _sc_gather(table, idx_flat).reshape(b, npoint * ns, cpad)
    cen = jnp.stack([ncx, ncy, ncz], axis=-1)
    cen = jnp.pad(cen, ((0, 0), (0, 0), (0, cpad - 3)))
    layers = [_prep_layer(l, cpad if i == 0 else None)
              for i, l in enumerate(layers_raw)]
    new_points = _sa_mlp_call(gathered, cen, layers, npoint, ns)
    return ncx, ncy, ncz, new_points


def kernel(point_clouds, params):
    b, n0, _ = point_clouds.shape
    x0 = point_clouds[:, :, 0]
    y0 = point_clouds[:, :, 1]
    z0 = point_clouds[:, :, 2]

    c1x, c1y, c1z, l1_points = _set_abstraction(
        x0, y0, z0, None, point_clouds, 1180, 0.2, 16,
        params['sa1'], n_real=n0, npad=4736, spad=1280, cpad=128)

    tab2 = jnp.concatenate(
        [jnp.stack([c1x, c1y, c1z], axis=-1), l1_points], axis=-1)
    c2x, c2y, c2z, l2_points = _set_abstraction(
        c1x, c1y, c1z, None, tab2, 295, 0.4, 16,
        params['sa2'], n_real=1180, npad=1280, spad=384, cpad=128)

    tab3 = jnp.concatenate(
        [jnp.stack([c2x, c2y, c2z], axis=-1), l2_points], axis=-1)
    c3x, c3y, c3z, l3_points = _set_abstraction(
        c2x, c2y, c2z, None, tab3, 59, 0.6, 16,
        params['sa3'], n_real=295, npad=384, spad=128, cpad=256)

    fp3_layers = [_prep_layer(l) for l in params['fp3']]
    x1 = [_pad_lanes(c, 384, "far")[:, :, None] for c in (c2x, c2y, c2z)]
    x2 = [_pad_lanes(c, 128, "far")[:, None, :] for c in (c3x, c3y, c3z)]
    p2 = _pad_rows(l3_points, 128)
    p1 = _pad_rows(l2_points, 384)
    fp3_out = _fp_call(x1, x2, p2, p1, fp3_layers, [True, True],
                       [True, True], 384)
    fp3_out = fp3_out[:, :295, :]

    fp2_layers = [_prep_layer(l) for l in params['fp2']]
    x1 = [_pad_lanes(c, 1280, "far")[:, :, None] for c in (c1x, c1y, c1z)]
    x2 = [_pad_lanes(c, 384, "far")[:, None, :] for c in (c2x, c2y, c2z)]
    p2 = _pad_rows(fp3_out, 384)
    p1 = _pad_rows(l1_points, 1280)
    fp2_out = _fp_call(x1, x2, p2, p1, fp2_layers, [True, True],
                       [True, True], 1280)
    fp2_out = fp2_out[:, :1180, :]

    fp1_layers = [_prep_layer(l) for l in params['fp1']]
    head1 = _prep_layer(params['head1'])
    w2, b2 = params['head2']
    ones1 = jnp.ones((1, 1), jnp.float32)
    zeros1 = jnp.zeros((1, 1), jnp.float32)
    head2 = (w2.T, b2[None, :], ones1, zeros1)
    layers = fp1_layers + [head1, head2]
    relu_flags = [True, True, True, True, False]
    bn_flags = [True, True, True, True, False]
    x1 = [_pad_lanes(c, 4736, "far")[:, :, None] for c in (x0, y0, z0)]
    x2 = [_pad_lanes(c, 1280, "far")[:, None, :] for c in (c1x, c1y, c1z)]
    p2 = _pad_rows(fp2_out, 1280)
    out = _fp_call(x1, x2, p2, None, layers, relu_flags, bn_flags, 592)
    return out[:, :n0, 0]
